# baseline (speedup 1.0000x reference)
import functools

import jax
import jax.numpy as jnp
from jax import lax
from jax.experimental import pallas as pl
from jax.experimental.pallas import tpu as pltpu
from jax.experimental.pallas import tpu_sc as plsc

NC = 2
NS = 16
NW = NC * NS
L = 16

BATCH = 16384
FEAT = 512
NBINS = 10240
BINS_PER_T = NBINS // NS
LABS_PER_T = BATCH // NS
ROWS_PER_W = BATCH // NW
CHUNK = 8
CHUNKS = ROWS_PER_W // CHUNK
RING = 4

_mesh = plsc.VectorSubcoreMesh(
    core_axis_name="c", subcore_axis_name="s", num_cores=NC, num_subcores=NS)
_params = pltpu.CompilerParams(needs_layout_passes=False)


@functools.partial(
    pl.kernel,
    out_type=(jax.ShapeDtypeStruct((NC, NS, NBINS), jnp.float32),
              jax.ShapeDtypeStruct((NC, NBINS), jnp.float32),
              jax.ShapeDtypeStruct((NW, L), jnp.float32),
              jax.ShapeDtypeStruct((10000, FEAT), jnp.float32)),
    mesh=_mesh,
    scratch_types=[
        pltpu.VMEM((LABS_PER_T,), jnp.float32),
        pltpu.VMEM((BINS_PER_T,), jnp.float32),
        pltpu.VMEM((ROWS_PER_W,), jnp.float32),
        pltpu.VMEM((ROWS_PER_W,), jnp.int32),
        pltpu.VMEM((NBINS,), jnp.float32),
        pltpu.VMEM((ROWS_PER_W,), jnp.float32),
        pltpu.VMEM((CHUNK, FEAT), jnp.float32),
        pltpu.VMEM((CHUNK, FEAT), jnp.float32),
        pltpu.VMEM((CHUNK, FEAT), jnp.float32),
        pltpu.VMEM((CHUNK, FEAT), jnp.float32),
        pltpu.VMEM((CHUNK, FEAT), jnp.float32),
        pltpu.VMEM((CHUNK, FEAT), jnp.float32),
        pltpu.VMEM((CHUNK, FEAT), jnp.float32),
        pltpu.VMEM((CHUNK, FEAT), jnp.float32),
        pltpu.VMEM((L,), jnp.float32),
        pltpu.SemaphoreType.DMA,
        pltpu.SemaphoreType.DMA,
        pltpu.SemaphoreType.DMA,
        pltpu.SemaphoreType.DMA,
        pltpu.SemaphoreType.DMA,
        pltpu.SemaphoreType.DMA,
        pltpu.SemaphoreType.DMA,
        pltpu.SemaphoreType.DMA,
        pltpu.SemaphoreType.DMA,
        pltpu.SemaphoreType.DMA,
    ],
    compiler_params=_params,
)
def _fused(f_hbm, lab_hbm, cen_hbm, hpart_hbm, cnt_hbm, out_hbm, cpy_hbm,
           hl_v, red_v, labf_v, lab_v, cnt_v, w_v,
           fb0, fb1, fb2, fb3, cb0, cb1, cb2, cb3, outb,
           semh, semcp,
           semf0, semf1, semf2, semf3, semc0, semc1, semc2, semc3):
    c = lax.axis_index("c")
    s = lax.axis_index("s")
    wid = s * NC + c
    base = wid * ROWS_PER_W
    lo = s * BINS_PER_T

    fbs = (fb0, fb1, fb2, fb3)
    cbs = (cb0, cb1, cb2, cb3)
    semf = (semf0, semf1, semf2, semf3)
    semc = (semc0, semc1, semc2, semc3)

    pltpu.sync_copy(lab_hbm.at[pl.ds(base, ROWS_PER_W)], labf_v)

    def _cv(j, carry):
        sl = pl.ds(j * L, L)
        lab_v[sl] = labf_v[sl].astype(jnp.int32)
        return carry

    lax.fori_loop(0, ROWS_PER_W // L, _cv, 0)

    def _startc(k, i):
        pltpu.async_copy(
            f_hbm.at[pl.ds(base + k * CHUNK, CHUNK)], fbs[i], semf[i])
        pltpu.async_copy(
            cen_hbm.at[lab_v.at[pl.ds(k * CHUNK, CHUNK)]], cbs[i], semc[i])

    for k in range(RING - 1):
        _startc(k, k)

    CROWS = 10000 // NW
    cpd = pltpu.async_copy(
        cen_hbm.at[pl.ds(wid * CROWS, CROWS)],
        cpy_hbm.at[pl.ds(wid * CROWS, CROWS)], semcp)
    cpd2 = None

    @pl.when(wid == NW - 1)
    def _():
        pltpu.async_copy(cen_hbm.at[pl.ds(NW * CROWS, 10000 - NW * CROWS)],
                         cpy_hbm.at[pl.ds(NW * CROWS, 10000 - NW * CROWS)],
                         semcp)

    hd = pltpu.async_copy(
        lab_hbm.at[pl.ds(s * LABS_PER_T, LABS_PER_T)], hl_v, semh)

    def _z(j, carry):
        cnt_v[pl.ds(j * L, L)] = jnp.zeros((L,), jnp.float32)
        return carry

    lax.fori_loop(0, NBINS // L, _z, 0)
    hd.wait()

    ones = jnp.ones((L,), jnp.float32)

    def _h(j, carry):
        lab = hl_v[pl.ds(j * L, L)].astype(jnp.int32)
        plsc.addupdate_scatter(cnt_v, [lab], ones)
        return carry

    lax.fori_loop(0, LABS_PER_T // L, _h, 0, unroll=4)

    pltpu.sync_copy(cnt_v, hpart_hbm.at[c, s])
    plsc.subcore_barrier()

    ds_ = []
    for t in range(NS):
        ds_.append(pltpu.async_copy(
            hpart_hbm.at[c, t, pl.ds(lo, BINS_PER_T)],
            cnt_v.at[pl.ds(t * BINS_PER_T, BINS_PER_T)], semh))
    for t in range(NS):
        ds_[t].wait()

    def _r(j, carry):
        sl = pl.ds(j * L, L)
        acc = None
        for t0 in range(0, NS, 4):
            a = (cnt_v[pl.ds((t0 + 0) * BINS_PER_T + j * L, L)]
                 + cnt_v[pl.ds((t0 + 1) * BINS_PER_T + j * L, L)])
            b = (cnt_v[pl.ds((t0 + 2) * BINS_PER_T + j * L, L)]
                 + cnt_v[pl.ds((t0 + 3) * BINS_PER_T + j * L, L)])
            acc = (a + b) if acc is None else acc + (a + b)
        red_v[sl] = acc
        return carry

    lax.fori_loop(0, BINS_PER_T // L, _r, 0)

    pltpu.sync_copy(red_v, cnt_hbm.at[c, pl.ds(lo, BINS_PER_T)])
    plsc.subcore_barrier()

    pltpu.sync_copy(cnt_hbm.at[c], cnt_v)

    def _w(j, carry):
        cnt = plsc.load_gather(cnt_v, [lab_v[pl.ds(j * L, L)]])
        w_v[pl.ds(j * L, L)] = 1.0 / cnt
        return carry

    lax.fori_loop(0, ROWS_PER_W // L, _w, 0)

    def _grp(g, gacc):
        for i in range(RING):
            k = g * RING + i
            q = k + RING - 1

            @pl.when(q < CHUNKS)
            def _():
                _startc(q, (i - 1) % RING)

            pltpu.make_async_copy(
                f_hbm.at[pl.ds(base, CHUNK)], fbs[i], semf[i]).wait()
            pltpu.make_async_copy(
                cen_hbm.at[lab_v.at[pl.ds(0, CHUNK)]], cbs[i], semc[i]).wait()
            fbuf = fbs[i]
            cbuf = cbs[i]

            def _row(r, gacc):
                accs = [jnp.zeros((L,), jnp.float32) for _ in range(4)]
                for j in range(FEAT // L):
                    dd = fbuf[r, pl.ds(j * L, L)] - cbuf[r, pl.ds(j * L, L)]
                    accs[j % 4] = accs[j % 4] + dd * dd
                acc = (accs[0] + accs[1]) + (accs[2] + accs[3])
                w = plsc.load_gather(
                    w_v, [jnp.full((L,), k * CHUNK, jnp.int32) + r])
                return gacc + acc * w

            gacc = lax.fori_loop(0, CHUNK, _row, gacc)
        return gacc

    gacc = lax.fori_loop(0, CHUNKS // RING, _grp,
                         jnp.zeros((L,), jnp.float32))

    outb[...] = gacc
    pltpu.sync_copy(outb, out_hbm.at[wid])
    cpd.wait()

    @pl.when(wid == NW - 1)
    def _():
        pltpu.make_async_copy(
            cen_hbm.at[pl.ds(NW * CROWS, 10000 - NW * CROWS)],
            cpy_hbm.at[pl.ds(NW * CROWS, 10000 - NW * CROWS)], semcp).wait()


def kernel(feature, labels, center, lamda):
    _, _, out, cen_out = _fused(feature, labels, center)
    loss = (lamda / 2) * (jnp.sum(out) / BATCH)
    return (loss, cen_out)


# speedup vs baseline: 8.6637x; 8.6637x over previous
import functools

import jax
import jax.numpy as jnp
from jax import lax
from jax.experimental import pallas as pl
from jax.experimental.pallas import tpu as pltpu
from jax.experimental.pallas import tpu_sc as plsc

NC = 2
NS = 16
NW = NC * NS
L = 16

BATCH = 16384
FEAT = 512
NBINS = 10240
BINS_PER_T = NBINS // NS
LABS_PER_T = BATCH // NS
ROWS_PER_W = BATCH // NW
CHUNK = 8
CHUNKS = ROWS_PER_W // CHUNK
RING = 4

_mesh = plsc.VectorSubcoreMesh(
    core_axis_name="c", subcore_axis_name="s", num_cores=NC, num_subcores=NS)
_params = pltpu.CompilerParams(needs_layout_passes=False)


@functools.partial(
    pl.kernel,
    out_type=(jax.ShapeDtypeStruct((NC, NS, NBINS), jnp.float32),
              jax.ShapeDtypeStruct((NC, NBINS), jnp.float32),
              jax.ShapeDtypeStruct((NW, L), jnp.float32)),
    mesh=_mesh,
    scratch_types=[
        pltpu.VMEM((LABS_PER_T,), jnp.float32),
        pltpu.VMEM((BINS_PER_T,), jnp.float32),
        pltpu.VMEM((ROWS_PER_W,), jnp.float32),
        pltpu.VMEM((ROWS_PER_W,), jnp.int32),
        pltpu.VMEM((NBINS,), jnp.float32),
        pltpu.VMEM((ROWS_PER_W,), jnp.float32),
        pltpu.VMEM((CHUNK, FEAT), jnp.float32),
        pltpu.VMEM((CHUNK, FEAT), jnp.float32),
        pltpu.VMEM((CHUNK, FEAT), jnp.float32),
        pltpu.VMEM((CHUNK, FEAT), jnp.float32),
        pltpu.VMEM((CHUNK, FEAT), jnp.float32),
        pltpu.VMEM((CHUNK, FEAT), jnp.float32),
        pltpu.VMEM((CHUNK, FEAT), jnp.float32),
        pltpu.VMEM((CHUNK, FEAT), jnp.float32),
        pltpu.VMEM((L,), jnp.float32),
        pltpu.SemaphoreType.DMA,
        pltpu.SemaphoreType.DMA,
        pltpu.SemaphoreType.DMA,
        pltpu.SemaphoreType.DMA,
        pltpu.SemaphoreType.DMA,
        pltpu.SemaphoreType.DMA,
        pltpu.SemaphoreType.DMA,
        pltpu.SemaphoreType.DMA,
        pltpu.SemaphoreType.DMA,
    ],
    compiler_params=_params,
)
def _fused(f_hbm, lab_hbm, cen_hbm, hpart_hbm, cnt_hbm, out_hbm,
           hl_v, red_v, labf_v, lab_v, cnt_v, w_v,
           fb0, fb1, fb2, fb3, cb0, cb1, cb2, cb3, outb,
           semh,
           semf0, semf1, semf2, semf3, semc0, semc1, semc2, semc3):
    c = lax.axis_index("c")
    s = lax.axis_index("s")
    wid = s * NC + c
    base = wid * ROWS_PER_W
    lo = s * BINS_PER_T

    fbs = (fb0, fb1, fb2, fb3)
    cbs = (cb0, cb1, cb2, cb3)
    semf = (semf0, semf1, semf2, semf3)
    semc = (semc0, semc1, semc2, semc3)

    pltpu.sync_copy(lab_hbm.at[pl.ds(base, ROWS_PER_W)], labf_v)

    def _cv(j, carry):
        sl = pl.ds(j * L, L)
        lab_v[sl] = labf_v[sl].astype(jnp.int32)
        return carry

    lax.fori_loop(0, ROWS_PER_W // L, _cv, 0)

    def _startc(k, i):
        pltpu.async_copy(
            f_hbm.at[pl.ds(base + k * CHUNK, CHUNK)], fbs[i], semf[i])
        pltpu.async_copy(
            cen_hbm.at[lab_v.at[pl.ds(k * CHUNK, CHUNK)]], cbs[i], semc[i])

    for k in range(RING - 1):
        _startc(k, k)

    hd = pltpu.async_copy(
        lab_hbm.at[pl.ds(s * LABS_PER_T, LABS_PER_T)], hl_v, semh)

    def _z(j, carry):
        cnt_v[pl.ds(j * L, L)] = jnp.zeros((L,), jnp.float32)
        return carry

    lax.fori_loop(0, NBINS // L, _z, 0)
    hd.wait()

    ones = jnp.ones((L,), jnp.float32)

    def _h(j, carry):
        lab = hl_v[pl.ds(j * L, L)].astype(jnp.int32)
        plsc.addupdate_scatter(cnt_v, [lab], ones)
        return carry

    lax.fori_loop(0, LABS_PER_T // L, _h, 0, unroll=4)

    pltpu.sync_copy(cnt_v, hpart_hbm.at[c, s])
    plsc.subcore_barrier()

    ds_ = []
    for t in range(NS):
        ds_.append(pltpu.async_copy(
            hpart_hbm.at[c, t, pl.ds(lo, BINS_PER_T)],
            cnt_v.at[pl.ds(t * BINS_PER_T, BINS_PER_T)], semh))
    for t in range(NS):
        ds_[t].wait()

    def _r(j, carry):
        sl = pl.ds(j * L, L)
        acc = None
        for t0 in range(0, NS, 4):
            a = (cnt_v[pl.ds((t0 + 0) * BINS_PER_T + j * L, L)]
                 + cnt_v[pl.ds((t0 + 1) * BINS_PER_T + j * L, L)])
            b = (cnt_v[pl.ds((t0 + 2) * BINS_PER_T + j * L, L)]
                 + cnt_v[pl.ds((t0 + 3) * BINS_PER_T + j * L, L)])
            acc = (a + b) if acc is None else acc + (a + b)
        red_v[sl] = acc
        return carry

    lax.fori_loop(0, BINS_PER_T // L, _r, 0)

    pltpu.sync_copy(red_v, cnt_hbm.at[c, pl.ds(lo, BINS_PER_T)])
    plsc.subcore_barrier()

    pltpu.sync_copy(cnt_hbm.at[c], cnt_v)

    def _w(j, carry):
        cnt = plsc.load_gather(cnt_v, [lab_v[pl.ds(j * L, L)]])
        w_v[pl.ds(j * L, L)] = 1.0 / cnt
        return carry

    lax.fori_loop(0, ROWS_PER_W // L, _w, 0)

    def _grp(g, gacc):
        for i in range(RING):
            k = g * RING + i
            q = k + RING - 1

            @pl.when(q < CHUNKS)
            def _():
                _startc(q, (i - 1) % RING)

            pltpu.make_async_copy(
                f_hbm.at[pl.ds(base, CHUNK)], fbs[i], semf[i]).wait()
            pltpu.make_async_copy(
                cen_hbm.at[lab_v.at[pl.ds(0, CHUNK)]], cbs[i], semc[i]).wait()
            fbuf = fbs[i]
            cbuf = cbs[i]

            def _row(r, gacc):
                accs = [jnp.zeros((L,), jnp.float32) for _ in range(4)]
                for j in range(FEAT // L):
                    dd = fbuf[r, pl.ds(j * L, L)] - cbuf[r, pl.ds(j * L, L)]
                    accs[j % 4] = accs[j % 4] + dd * dd
                acc = (accs[0] + accs[1]) + (accs[2] + accs[3])
                w = plsc.load_gather(
                    w_v, [jnp.full((L,), k * CHUNK, jnp.int32) + r])
                return gacc + acc * w

            gacc = lax.fori_loop(0, CHUNK, _row, gacc, unroll=2)
        return gacc

    gacc = lax.fori_loop(0, CHUNKS // RING, _grp,
                         jnp.zeros((L,), jnp.float32))

    outb[...] = gacc
    pltpu.sync_copy(outb, out_hbm.at[wid])


def kernel(feature, labels, center, lamda):
    _, _, out = _fused(feature, labels, center)
    loss = (lamda / 2) * (jnp.sum(out) / BATCH)
    return (loss, center)


# speedup vs baseline: 8.8521x; 1.0218x over previous
import functools

import jax
import jax.numpy as jnp
from jax import lax
from jax.experimental import pallas as pl
from jax.experimental.pallas import tpu as pltpu
from jax.experimental.pallas import tpu_sc as plsc

NC = 2
NS = 16
NW = NC * NS
L = 16

BATCH = 16384
FEAT = 512
NBINS = 10240
BINS_PER_T = NBINS // NS
LABS_PER_T = BATCH // NS
ROWS_PER_W = BATCH // NW
CHUNK = 8
CHUNKS = ROWS_PER_W // CHUNK
RING = 4

_mesh = plsc.VectorSubcoreMesh(
    core_axis_name="c", subcore_axis_name="s", num_cores=NC, num_subcores=NS)
_params = pltpu.CompilerParams(needs_layout_passes=False)


@functools.partial(
    pl.kernel,
    out_type=(jax.ShapeDtypeStruct((NC, NS, NBINS), jnp.float32),
              jax.ShapeDtypeStruct((NC, NBINS), jnp.float32),
              jax.ShapeDtypeStruct((NW, L), jnp.float32)),
    mesh=_mesh,
    scratch_types=[
        pltpu.VMEM((LABS_PER_T,), jnp.float32),
        pltpu.VMEM((BINS_PER_T,), jnp.float32),
        pltpu.VMEM((ROWS_PER_W,), jnp.float32),
        pltpu.VMEM((ROWS_PER_W,), jnp.int32),
        pltpu.VMEM((NBINS,), jnp.float32),
        pltpu.VMEM((ROWS_PER_W,), jnp.float32),
        pltpu.VMEM((CHUNK, FEAT), jnp.float32),
        pltpu.VMEM((CHUNK, FEAT), jnp.float32),
        pltpu.VMEM((CHUNK, FEAT), jnp.float32),
        pltpu.VMEM((CHUNK, FEAT), jnp.float32),
        pltpu.VMEM((CHUNK, FEAT), jnp.float32),
        pltpu.VMEM((CHUNK, FEAT), jnp.float32),
        pltpu.VMEM((CHUNK, FEAT), jnp.float32),
        pltpu.VMEM((CHUNK, FEAT), jnp.float32),
        pltpu.VMEM((L,), jnp.float32),
        pltpu.SemaphoreType.DMA,
        pltpu.SemaphoreType.DMA,
        pltpu.SemaphoreType.DMA,
        pltpu.SemaphoreType.DMA,
        pltpu.SemaphoreType.DMA,
        pltpu.SemaphoreType.DMA,
        pltpu.SemaphoreType.DMA,
        pltpu.SemaphoreType.DMA,
        pltpu.SemaphoreType.DMA,
    ],
    compiler_params=_params,
)
def _fused(f_hbm, lab_hbm, cen_hbm, hpart_hbm, cnt_hbm, out_hbm,
           hl_v, red_v, labf_v, lab_v, cnt_v, w_v,
           fb0, fb1, fb2, fb3, cb0, cb1, cb2, cb3, outb,
           semh,
           semf0, semf1, semf2, semf3, semc0, semc1, semc2, semc3):
    c = lax.axis_index("c")
    s = lax.axis_index("s")
    wid = s * NC + c
    base = wid * ROWS_PER_W
    lo = s * BINS_PER_T

    fbs = (fb0, fb1, fb2, fb3)
    cbs = (cb0, cb1, cb2, cb3)
    semf = (semf0, semf1, semf2, semf3)
    semc = (semc0, semc1, semc2, semc3)

    pltpu.sync_copy(lab_hbm.at[pl.ds(base, ROWS_PER_W)], labf_v)

    def _cv(j, carry):
        sl = pl.ds(j * L, L)
        lab_v[sl] = labf_v[sl].astype(jnp.int32)
        return carry

    lax.fori_loop(0, ROWS_PER_W // L, _cv, 0)

    def _startc(k, i):
        pltpu.async_copy(
            f_hbm.at[pl.ds(base + k * CHUNK, CHUNK)], fbs[i], semf[i])
        pltpu.async_copy(
            cen_hbm.at[lab_v.at[pl.ds(k * CHUNK, CHUNK)]], cbs[i], semc[i])

    for k in range(RING - 1):
        _startc(k, k)

    hd = pltpu.async_copy(
        lab_hbm.at[pl.ds(s * LABS_PER_T, LABS_PER_T)], hl_v, semh)

    def _z(j, carry):
        cnt_v[pl.ds(j * L, L)] = jnp.zeros((L,), jnp.float32)
        return carry

    lax.fori_loop(0, NBINS // L, _z, 0)
    hd.wait()

    ones = jnp.ones((L,), jnp.float32)

    def _h(j, carry):
        lab = hl_v[pl.ds(j * L, L)].astype(jnp.int32)
        plsc.addupdate_scatter(cnt_v, [lab], ones)
        return carry

    lax.fori_loop(0, LABS_PER_T // L, _h, 0, unroll=4)

    pltpu.sync_copy(cnt_v, hpart_hbm.at[c, s])
    plsc.subcore_barrier()

    ds_ = []
    for t in range(NS):
        ds_.append(pltpu.async_copy(
            hpart_hbm.at[c, t, pl.ds(lo, BINS_PER_T)],
            cnt_v.at[pl.ds(t * BINS_PER_T, BINS_PER_T)], semh))
    for t in range(NS):
        ds_[t].wait()

    def _r(j, carry):
        sl = pl.ds(j * L, L)
        acc = None
        for t0 in range(0, NS, 4):
            a = (cnt_v[pl.ds((t0 + 0) * BINS_PER_T + j * L, L)]
                 + cnt_v[pl.ds((t0 + 1) * BINS_PER_T + j * L, L)])
            b = (cnt_v[pl.ds((t0 + 2) * BINS_PER_T + j * L, L)]
                 + cnt_v[pl.ds((t0 + 3) * BINS_PER_T + j * L, L)])
            acc = (a + b) if acc is None else acc + (a + b)
        red_v[sl] = acc
        return carry

    lax.fori_loop(0, BINS_PER_T // L, _r, 0)

    pltpu.sync_copy(red_v, cnt_hbm.at[c, pl.ds(lo, BINS_PER_T)])
    plsc.subcore_barrier()

    pltpu.sync_copy(cnt_hbm.at[c], cnt_v)

    def _w(j, carry):
        cnt = plsc.load_gather(cnt_v, [lab_v[pl.ds(j * L, L)]])
        w_v[pl.ds(j * L, L)] = 1.0 / cnt
        return carry

    lax.fori_loop(0, ROWS_PER_W // L, _w, 0)

    def _grp(g, gacc):
        for i in range(RING):
            k = g * RING + i
            q = k + RING - 1

            @pl.when(q < CHUNKS)
            def _():
                _startc(q, (i - 1) % RING)

            pltpu.make_async_copy(
                f_hbm.at[pl.ds(base, CHUNK)], fbs[i], semf[i]).wait()
            pltpu.make_async_copy(
                cen_hbm.at[lab_v.at[pl.ds(0, CHUNK)]], cbs[i], semc[i]).wait()
            fbuf = fbs[i]
            cbuf = cbs[i]

            def _row(r, gacc):
                accs = [jnp.zeros((L,), jnp.float32) for _ in range(4)]
                for j in range(FEAT // L):
                    dd = fbuf[r, pl.ds(j * L, L)] - cbuf[r, pl.ds(j * L, L)]
                    accs[j % 4] = accs[j % 4] + dd * dd
                acc = (accs[0] + accs[1]) + (accs[2] + accs[3])
                w = plsc.load_gather(
                    w_v, [jnp.full((L,), k * CHUNK, jnp.int32) + r])
                return gacc + acc * w

            gacc = lax.fori_loop(0, CHUNK, _row, gacc)
        return gacc

    gacc = lax.fori_loop(0, CHUNKS // RING, _grp,
                         jnp.zeros((L,), jnp.float32))

    outb[...] = gacc
    pltpu.sync_copy(outb, out_hbm.at[wid])


def kernel(feature, labels, center, lamda):
    _, _, out = _fused(feature, labels, center)
    loss = (lamda / 2) * (jnp.sum(out) / BATCH)
    return (loss, center)
